# Initial kernel scaffold; baseline (speedup 1.0000x reference)
import jax
import jax.numpy as jnp
from jax.experimental import pallas as pl


def kernel(edge_index, h, e, Wq, bq, Wk, bk, Wv, bv, We, be):
    raise NotImplementedError("write your pallas kernel here")


# speedup vs baseline: 14.2199x; 14.2199x over previous
import jax
import jax.numpy as jnp
from jax import lax
from jax.experimental import pallas as pl
from jax.experimental.pallas import tpu as pltpu
from jax.experimental.pallas import tpu_sc as plsc

_N = 10000
_E = 320000
_H = 8
_D = 16
_HD = _H * _D

_NC = 2
_NS = 16
_NW = _NC * _NS
_EPW = _E // _NW
_C = 40
_NCHUNK = _EPW // _C
_ACC_W = 144
_RPW = _N // _NS
_ZR = 25


def _sc_attn_body(src_hbm, dst_hbm, q_hbm, kv_hbm, pe_hbm,
                  eout_hbm, acc_hbm,
                  src_v, dst_v, q_v, kv_v, pe_v, eout_v, contrib_v, zbuf_v,
                  acc_sh, sem_kv, sem_q):
    c = lax.axis_index("c")
    s = lax.axis_index("s")
    wid = c * _NS + s
    ebase = wid * _EPW

    zeros16 = jnp.zeros((16,), jnp.float32)

    def zrow(r, carry):
        for cc in range(_ACC_W // 16):
            zbuf_v[r, pl.ds(cc * 16, 16)] = zeros16
        return carry

    lax.fori_loop(0, _ZR, zrow, 0)
    for j in range(_RPW // _ZR):
        pltpu.sync_copy(zbuf_v, acc_sh.at[pl.ds(s * _RPW + j * _ZR, _ZR)])
    plsc.subcore_barrier()

    lane = lax.iota(jnp.int32, 16)

    def chunk(i, carry):
        base = ebase + i * _C
        pltpu.sync_copy(src_hbm.at[pl.ds(base, _C)], src_v)
        pltpu.sync_copy(dst_hbm.at[pl.ds(base, _C)], dst_v)
        cp_kv = pltpu.async_copy(kv_hbm.at[src_v], kv_v, sem_kv)
        cp_q = pltpu.async_copy(q_hbm.at[dst_v], q_v, sem_q)
        pltpu.sync_copy(pe_hbm.at[pl.ds(base, _C)], pe_v)
        cp_kv.wait()
        cp_q.wait()

        def edge(j, ecarry):
            zvec = jnp.zeros((16,), jnp.float32)
            for hd in range(_H):
                o = hd * 16
                qv = q_v[j, pl.ds(o, 16)]
                kv = kv_v[j, pl.ds(o, 16)]
                vv = kv_v[j, pl.ds(_HD + o, 16)]
                pev = pe_v[j, pl.ds(o, 16)]
                score = (kv * qv) * pev * 0.25
                eout_v[j, pl.ds(o, 16)] = score
                t = jnp.broadcast_to(jnp.sum(score), (16,))
                sv = jnp.exp(jnp.clip(t, -5.0, 5.0))
                contrib_v[j, pl.ds(o, 16)] = vv * sv
                zvec = jnp.where(lane == hd, sv, zvec)
            contrib_v[j, pl.ds(_HD, 16)] = zvec
            return ecarry

        lax.fori_loop(0, _C, edge, 0)

        pltpu.sync_copy(eout_v, eout_hbm.at[pl.ds(base, _C)])
        pltpu.sync_copy(contrib_v, acc_sh.at[dst_v], add=True)
        return carry

    lax.fori_loop(0, _NCHUNK, chunk, 0)

    plsc.subcore_barrier()
    pltpu.sync_copy(acc_sh.at[pl.ds(s * _RPW, _RPW)],
                    acc_hbm.at[c, pl.ds(s * _RPW, _RPW)])


_sc_attn = pl.kernel(
    _sc_attn_body,
    out_type=[
        jax.ShapeDtypeStruct((_E, _HD), jnp.float32),
        jax.ShapeDtypeStruct((_NC, _N, _ACC_W), jnp.float32),
    ],
    mesh=plsc.VectorSubcoreMesh(core_axis_name="c", subcore_axis_name="s"),
    compiler_params=pltpu.CompilerParams(use_tc_tiling_on_sc=False,
                                         needs_layout_passes=False),
    scratch_types=[
        pltpu.VMEM((_C,), jnp.int32),
        pltpu.VMEM((_C,), jnp.int32),
        pltpu.VMEM((_C, _HD), jnp.float32),
        pltpu.VMEM((_C, 2 * _HD), jnp.float32),
        pltpu.VMEM((_C, _HD), jnp.float32),
        pltpu.VMEM((_C, _HD), jnp.float32),
        pltpu.VMEM((_C, _ACC_W), jnp.float32),
        pltpu.VMEM((_ZR, _ACC_W), jnp.float32),
        pltpu.VMEM_SHARED((_N, _ACC_W), jnp.float32),
        pltpu.SemaphoreType.DMA,
        pltpu.SemaphoreType.DMA,
    ],
)


_TB = 2000


def _tables_body(h_ref, wq_ref, bq_ref, wkv_ref, bkv_ref, q_out, kv_out):
    hb = h_ref[...]
    q_out[...] = jnp.dot(hb, wq_ref[...], preferred_element_type=jnp.float32,
                         precision=lax.Precision.HIGHEST) + bq_ref[...]
    kv_out[...] = jnp.dot(hb, wkv_ref[...], preferred_element_type=jnp.float32,
                          precision=lax.Precision.HIGHEST) + bkv_ref[...]


_tables = pl.pallas_call(
    _tables_body,
    grid=(_N // _TB,),
    in_specs=[
        pl.BlockSpec((_TB, _HD), lambda i: (i, 0)),
        pl.BlockSpec((_HD, _HD), lambda i: (0, 0)),
        pl.BlockSpec((1, _HD), lambda i: (0, 0)),
        pl.BlockSpec((_HD, 2 * _HD), lambda i: (0, 0)),
        pl.BlockSpec((1, 2 * _HD), lambda i: (0, 0)),
    ],
    out_specs=[
        pl.BlockSpec((_TB, _HD), lambda i: (i, 0)),
        pl.BlockSpec((_TB, 2 * _HD), lambda i: (i, 0)),
    ],
    out_shape=[
        jax.ShapeDtypeStruct((_N, _HD), jnp.float32),
        jax.ShapeDtypeStruct((_N, 2 * _HD), jnp.float32),
    ],
)

_EB = 2000


def _pe_body(e_ref, we_ref, be_ref, out_ref):
    out_ref[...] = jnp.dot(e_ref[...], we_ref[...],
                           preferred_element_type=jnp.float32,
                           precision=lax.Precision.HIGHEST) + be_ref[...]


_pe = pl.pallas_call(
    _pe_body,
    grid=(_E // _EB,),
    in_specs=[
        pl.BlockSpec((_EB, _HD), lambda i: (i, 0)),
        pl.BlockSpec((_HD, _HD), lambda i: (0, 0)),
        pl.BlockSpec((1, _HD), lambda i: (0, 0)),
    ],
    out_specs=pl.BlockSpec((_EB, _HD), lambda i: (i, 0)),
    out_shape=jax.ShapeDtypeStruct((_E, _HD), jnp.float32),
)

_CB = 2000


def _combine_body(a0_ref, a1_ref, r_ref, out_ref):
    wv = a0_ref[:, : _HD] + a1_ref[:, : _HD]
    z = a0_ref[:, _HD:_HD + _H] + a1_ref[:, _HD:_HD + _H]
    zr = lax.dot_general(z, r_ref[...], (((1,), (0,)), ((), ())),
                         precision=lax.Precision.HIGHEST,
                         preferred_element_type=jnp.float32)
    out_ref[...] = wv / (zr + 1e-6)


_combine = pl.pallas_call(
    _combine_body,
    grid=(_N // _CB,),
    in_specs=[
        pl.BlockSpec((_CB, _ACC_W), lambda i: (i, 0)),
        pl.BlockSpec((_CB, _ACC_W), lambda i: (i, 0)),
        pl.BlockSpec((_H, _HD), lambda i: (0, 0)),
    ],
    out_specs=pl.BlockSpec((_CB, _HD), lambda i: (i, 0)),
    out_shape=jax.ShapeDtypeStruct((_N, _HD), jnp.float32),
)


def kernel(edge_index, h, e, Wq, bq, Wk, bk, Wv, bv, We, be):
    src = edge_index[0].astype(jnp.int32)
    dst = edge_index[1].astype(jnp.int32)
    Wkv = jnp.concatenate([Wk, Wv], axis=1)
    bkv = jnp.concatenate([bk, bv])
    qtab, kvtab = _tables(h, Wq, bq.reshape(1, -1), Wkv, bkv.reshape(1, -1))
    pe = _pe(e, We, be.reshape(1, -1))
    eout, acc = _sc_attn(src, dst, qtab, kvtab, pe)
    r = jnp.repeat(jnp.eye(_H, dtype=jnp.float32), _D, axis=1)
    hout = _combine(acc[0], acc[1], r)
    return hout.reshape(_N, _H, _D), eout.reshape(_E, _H, _D)


# speedup vs baseline: 31.5295x; 2.2173x over previous
import jax
import jax.numpy as jnp
from jax import lax
from jax.experimental import pallas as pl
from jax.experimental.pallas import tpu as pltpu
from jax.experimental.pallas import tpu_sc as plsc

_N = 10000
_E = 320000
_H = 8
_D = 16
_HD = _H * _D

_NC = 2
_NS = 16
_NW = _NC * _NS
_EPW = _E // _NW
_C = 40
_NCHUNK = _EPW // _C
_ACC_W = 144
_RPW = _N // _NS
_ZR = 25


def _sc_attn_body(src_hbm, dst_hbm, q_hbm, kv_hbm, pe_hbm,
                  eout_hbm, acc_hbm,
                  src_v, dst_v, q_v, kv_v, pe_v, eout_v, contrib_v, zbuf_v,
                  acc_sh, sem_kv, sem_q):
    c = lax.axis_index("c")
    s = lax.axis_index("s")
    wid = c * _NS + s
    ebase = wid * _EPW

    zeros16 = jnp.zeros((16,), jnp.float32)

    def zrow(r, carry):
        for cc in range(_ACC_W // 16):
            zbuf_v[r, pl.ds(cc * 16, 16)] = zeros16
        return carry

    lax.fori_loop(0, _ZR, zrow, 0)
    for j in range(_RPW // _ZR):
        pltpu.sync_copy(zbuf_v, acc_sh.at[pl.ds(s * _RPW + j * _ZR, _ZR)])
    plsc.subcore_barrier()

    lane = lax.iota(jnp.int32, 16)

    def chunk(i, carry):
        base = ebase + i * _C
        pltpu.sync_copy(src_hbm.at[pl.ds(base, _C)], src_v)
        pltpu.sync_copy(dst_hbm.at[pl.ds(base, _C)], dst_v)
        cp_kv = pltpu.async_copy(kv_hbm.at[src_v], kv_v, sem_kv)
        cp_q = pltpu.async_copy(q_hbm.at[dst_v], q_v, sem_q)
        pltpu.sync_copy(pe_hbm.at[pl.ds(base, _C)], pe_v)
        cp_kv.wait()
        cp_q.wait()

        @plsc.parallel_loop(0, _C, unroll=2)
        def edge(j):
            zvec = jnp.zeros((16,), jnp.float32)
            for hd in range(_H):
                o = hd * 16
                qv = q_v[j, pl.ds(o, 16)]
                kv = kv_v[j, pl.ds(o, 16)]
                vv = kv_v[j, pl.ds(_HD + o, 16)]
                pev = pe_v[j, pl.ds(o, 16)]
                score = (kv * qv) * pev
                eout_v[j, pl.ds(o, 16)] = score
                t = jnp.broadcast_to(jnp.sum(score), (16,))
                sv = jnp.exp(jnp.clip(t, -5.0, 5.0))
                contrib_v[j, pl.ds(o, 16)] = vv * sv
                zvec = jnp.where(lane == hd, sv, zvec)
            contrib_v[j, pl.ds(_HD, 16)] = zvec

        pltpu.sync_copy(eout_v, eout_hbm.at[pl.ds(base, _C)])
        pltpu.sync_copy(contrib_v, acc_sh.at[dst_v], add=True)
        return carry

    lax.fori_loop(0, _NCHUNK, chunk, 0)

    plsc.subcore_barrier()
    pltpu.sync_copy(acc_sh.at[pl.ds(s * _RPW, _RPW)],
                    acc_hbm.at[c, pl.ds(s * _RPW, _RPW)])


_sc_attn = pl.kernel(
    _sc_attn_body,
    out_type=[
        jax.ShapeDtypeStruct((_E, _HD), jnp.float32),
        jax.ShapeDtypeStruct((_NC, _N, _ACC_W), jnp.float32),
    ],
    mesh=plsc.VectorSubcoreMesh(core_axis_name="c", subcore_axis_name="s"),
    compiler_params=pltpu.CompilerParams(use_tc_tiling_on_sc=False,
                                         needs_layout_passes=False),
    scratch_types=[
        pltpu.VMEM((_C,), jnp.int32),
        pltpu.VMEM((_C,), jnp.int32),
        pltpu.VMEM((_C, _HD), jnp.float32),
        pltpu.VMEM((_C, 2 * _HD), jnp.float32),
        pltpu.VMEM((_C, _HD), jnp.float32),
        pltpu.VMEM((_C, _HD), jnp.float32),
        pltpu.VMEM((_C, _ACC_W), jnp.float32),
        pltpu.VMEM((_ZR, _ACC_W), jnp.float32),
        pltpu.VMEM_SHARED((_N, _ACC_W), jnp.float32),
        pltpu.SemaphoreType.DMA,
        pltpu.SemaphoreType.DMA,
    ],
)


_TB = 2000


def _tables_body(h_ref, wq_ref, bq_ref, wkv_ref, bkv_ref, q_out, kv_out):
    hb = h_ref[...]
    q_out[...] = jnp.dot(hb, wq_ref[...], preferred_element_type=jnp.float32,
                         precision=lax.Precision.HIGHEST) + bq_ref[...]
    kv_out[...] = jnp.dot(hb, wkv_ref[...], preferred_element_type=jnp.float32,
                          precision=lax.Precision.HIGHEST) + bkv_ref[...]


_tables = pl.pallas_call(
    _tables_body,
    grid=(_N // _TB,),
    in_specs=[
        pl.BlockSpec((_TB, _HD), lambda i: (i, 0)),
        pl.BlockSpec((_HD, _HD), lambda i: (0, 0)),
        pl.BlockSpec((1, _HD), lambda i: (0, 0)),
        pl.BlockSpec((_HD, 2 * _HD), lambda i: (0, 0)),
        pl.BlockSpec((1, 2 * _HD), lambda i: (0, 0)),
    ],
    out_specs=[
        pl.BlockSpec((_TB, _HD), lambda i: (i, 0)),
        pl.BlockSpec((_TB, 2 * _HD), lambda i: (i, 0)),
    ],
    out_shape=[
        jax.ShapeDtypeStruct((_N, _HD), jnp.float32),
        jax.ShapeDtypeStruct((_N, 2 * _HD), jnp.float32),
    ],
)

_EB = 2000


def _pe_body(e_ref, we_ref, be_ref, out_ref):
    out_ref[...] = (jnp.dot(e_ref[...], we_ref[...],
                            preferred_element_type=jnp.float32,
                            precision=lax.Precision.HIGHEST) * 0.25
                    + be_ref[...] * 0.25)


_pe = pl.pallas_call(
    _pe_body,
    grid=(_E // _EB,),
    in_specs=[
        pl.BlockSpec((_EB, _HD), lambda i: (i, 0)),
        pl.BlockSpec((_HD, _HD), lambda i: (0, 0)),
        pl.BlockSpec((1, _HD), lambda i: (0, 0)),
    ],
    out_specs=pl.BlockSpec((_EB, _HD), lambda i: (i, 0)),
    out_shape=jax.ShapeDtypeStruct((_E, _HD), jnp.float32),
)

_CB = 2000


def _combine_body(a0_ref, a1_ref, r_ref, out_ref):
    wv = a0_ref[:, : _HD] + a1_ref[:, : _HD]
    z = a0_ref[:, _HD:_HD + _H] + a1_ref[:, _HD:_HD + _H]
    zr = lax.dot_general(z, r_ref[...], (((1,), (0,)), ((), ())),
                         precision=lax.Precision.HIGHEST,
                         preferred_element_type=jnp.float32)
    out_ref[...] = wv / (zr + 1e-6)


_combine = pl.pallas_call(
    _combine_body,
    grid=(_N // _CB,),
    in_specs=[
        pl.BlockSpec((_CB, _ACC_W), lambda i: (i, 0)),
        pl.BlockSpec((_CB, _ACC_W), lambda i: (i, 0)),
        pl.BlockSpec((_H, _HD), lambda i: (0, 0)),
    ],
    out_specs=pl.BlockSpec((_CB, _HD), lambda i: (i, 0)),
    out_shape=jax.ShapeDtypeStruct((_N, _HD), jnp.float32),
)


def kernel(edge_index, h, e, Wq, bq, Wk, bk, Wv, bv, We, be):
    src = edge_index[0].astype(jnp.int32)
    dst = edge_index[1].astype(jnp.int32)
    Wkv = jnp.concatenate([Wk, Wv], axis=1)
    bkv = jnp.concatenate([bk, bv])
    qtab, kvtab = _tables(h, Wq, bq.reshape(1, -1), Wkv, bkv.reshape(1, -1))
    pe = _pe(e, We, be.reshape(1, -1))
    eout, acc = _sc_attn(src, dst, qtab, kvtab, pe)
    r = jnp.repeat(jnp.eye(_H, dtype=jnp.float32), _D, axis=1)
    hout = _combine(acc[0], acc[1], r)
    return hout.reshape(_N, _H, _D), eout.reshape(_E, _H, _D)
